# baseline (speedup 1.0000x reference)
import functools

import jax
import jax.numpy as jnp
from jax import lax
from jax.experimental import pallas as pl
from jax.experimental.pallas import tpu as pltpu
from jax.experimental.pallas import tpu_sc as plsc

NUM_STEPS = 1000
BATCH = 16384
NS = 16
LANES = 16
B_PER_W = BATCH // NS
NCHUNK = 4
CSZ = B_PER_W // NCHUNK


@functools.partial(
    pl.kernel,
    mesh=plsc.VectorSubcoreMesh(core_axis_name="c", subcore_axis_name="s",
                                num_cores=1),
    compiler_params=pltpu.CompilerParams(needs_layout_passes=False),
    out_type=(
        jax.ShapeDtypeStruct((BATCH,), jnp.float32),
        jax.ShapeDtypeStruct((BATCH,), jnp.float32),
    ),
    scratch_types=[
        pltpu.VMEM((B_PER_W,), jnp.int32),
        pltpu.VMEM((NUM_STEPS,), jnp.float32),
        pltpu.VMEM((NUM_STEPS,), jnp.float32),
        pltpu.VMEM((B_PER_W,), jnp.float32),
        pltpu.VMEM((B_PER_W,), jnp.float32),
        [pltpu.SemaphoreType.DMA] * NCHUNK,
        pltpu.SemaphoreType.DMA,
        pltpu.SemaphoreType.DMA,
    ],
)
def _gather_sc(t_hbm, beta_hbm, alpha_hbm, alpha_out, beta_out,
               idx_v, beta_v, alpha_v, oa_v, ob_v, sem_i, sem_t, sem_o):
    wid = lax.axis_index("s")
    base = wid * B_PER_W
    cp_i = [
        pltpu.make_async_copy(t_hbm.at[pl.ds(base + c * CSZ, CSZ)],
                              idx_v.at[pl.ds(c * CSZ, CSZ)], sem_i[c])
        for c in range(NCHUNK)
    ]
    cp_ta = pltpu.make_async_copy(alpha_hbm, alpha_v, sem_t)
    cp_tb = pltpu.make_async_copy(beta_hbm, beta_v, sem_t)
    cp_i[0].start()
    cp_ta.start()
    cp_tb.start()
    for c in range(1, NCHUNK):
        cp_i[c].start()
    cp_ta.wait()
    cp_tb.wait()
    outs = []
    for c in range(NCHUNK):
        cp_i[c].wait()
        for i in range(c * (CSZ // LANES), (c + 1) * (CSZ // LANES)):
            sl = pl.ds(i * LANES, LANES)
            idx = idx_v[sl]
            oa_v[sl] = plsc.load_gather(alpha_v, [idx])
            ob_v[sl] = plsc.load_gather(beta_v, [idx])
        cp_oa = pltpu.make_async_copy(
            oa_v.at[pl.ds(c * CSZ, CSZ)],
            alpha_out.at[pl.ds(base + c * CSZ, CSZ)], sem_o)
        cp_ob = pltpu.make_async_copy(
            ob_v.at[pl.ds(c * CSZ, CSZ)],
            beta_out.at[pl.ds(base + c * CSZ, CSZ)], sem_o)
        cp_oa.start()
        cp_ob.start()
        outs.append((cp_oa, cp_ob))
    for cp_oa, cp_ob in outs:
        cp_oa.wait()
        cp_ob.wait()


def kernel(t, beta_t, alpha_t):
    alpha_g, beta_g = _gather_sc(t.astype(jnp.int32),
                                 beta_t.astype(jnp.float32),
                                 alpha_t.astype(jnp.float32))
    return alpha_g, beta_g


# speedup vs baseline: 1.0112x; 1.0112x over previous
import functools

import jax
import jax.numpy as jnp
from jax import lax
from jax.experimental import pallas as pl
from jax.experimental.pallas import tpu as pltpu
from jax.experimental.pallas import tpu_sc as plsc

NUM_STEPS = 1000
BATCH = 16384
NC = 1
NS = 16
NW = NC * NS
LANES = 16
B_PER_W = BATCH // NW
HALF = B_PER_W // 2


@functools.partial(
    pl.kernel,
    mesh=plsc.VectorSubcoreMesh(core_axis_name="c", subcore_axis_name="s", num_cores=1),
    compiler_params=pltpu.CompilerParams(needs_layout_passes=False),
    out_type=(
        jax.ShapeDtypeStruct((BATCH,), jnp.float32),
        jax.ShapeDtypeStruct((BATCH,), jnp.float32),
    ),
    scratch_types=[
        pltpu.VMEM((B_PER_W,), jnp.int32),
        pltpu.VMEM((NUM_STEPS,), jnp.float32),
        pltpu.VMEM((NUM_STEPS,), jnp.float32),
        pltpu.VMEM((B_PER_W,), jnp.float32),
        pltpu.VMEM((B_PER_W,), jnp.float32),
        pltpu.SemaphoreType.DMA,
        pltpu.SemaphoreType.DMA,
        pltpu.SemaphoreType.DMA,
        pltpu.SemaphoreType.DMA,
    ],
)
def _gather_sc(t_hbm, beta_hbm, alpha_hbm, alpha_out, beta_out,
               idx_v, beta_v, alpha_v, oa_v, ob_v,
               sem_i0, sem_i1, sem_t, sem_o):
    wid = lax.axis_index("s")
    base = wid * B_PER_W
    cp_i0 = pltpu.make_async_copy(t_hbm.at[pl.ds(base, HALF)],
                                  idx_v.at[pl.ds(0, HALF)], sem_i0)
    cp_i1 = pltpu.make_async_copy(t_hbm.at[pl.ds(base + HALF, HALF)],
                                  idx_v.at[pl.ds(HALF, HALF)], sem_i1)
    cp_ta = pltpu.make_async_copy(alpha_hbm, alpha_v, sem_t)
    cp_tb = pltpu.make_async_copy(beta_hbm, beta_v, sem_t)
    cp_i0.start()
    cp_ta.start()
    cp_tb.start()
    cp_i1.start()
    cp_i0.wait()
    cp_ta.wait()
    cp_tb.wait()
    for i in range(HALF // LANES):
        sl = pl.ds(i * LANES, LANES)
        idx = idx_v[sl]
        oa_v[sl] = plsc.load_gather(alpha_v, [idx])
        ob_v[sl] = plsc.load_gather(beta_v, [idx])
    cp_oa0 = pltpu.make_async_copy(oa_v.at[pl.ds(0, HALF)],
                                   alpha_out.at[pl.ds(base, HALF)], sem_o)
    cp_ob0 = pltpu.make_async_copy(ob_v.at[pl.ds(0, HALF)],
                                   beta_out.at[pl.ds(base, HALF)], sem_o)
    cp_oa0.start()
    cp_ob0.start()
    cp_i1.wait()
    for i in range(HALF // LANES, B_PER_W // LANES):
        sl = pl.ds(i * LANES, LANES)
        idx = idx_v[sl]
        oa_v[sl] = plsc.load_gather(alpha_v, [idx])
        ob_v[sl] = plsc.load_gather(beta_v, [idx])
    cp_oa1 = pltpu.make_async_copy(oa_v.at[pl.ds(HALF, HALF)],
                                   alpha_out.at[pl.ds(base + HALF, HALF)],
                                   sem_o)
    cp_ob1 = pltpu.make_async_copy(ob_v.at[pl.ds(HALF, HALF)],
                                   beta_out.at[pl.ds(base + HALF, HALF)],
                                   sem_o)
    cp_oa1.start()
    cp_ob1.start()
    cp_oa0.wait()
    cp_ob0.wait()
    cp_oa1.wait()
    cp_ob1.wait()


def kernel(t, beta_t, alpha_t):
    alpha_g, beta_g = _gather_sc(t.astype(jnp.int32),
                                 beta_t.astype(jnp.float32),
                                 alpha_t.astype(jnp.float32))
    return alpha_g, beta_g
